# baseline (speedup 1.0000x reference)
import functools

import jax
import jax.numpy as jnp
from jax import lax
from jax.experimental import pallas as pl
from jax.experimental.pallas import tpu as pltpu
from jax.experimental.pallas import tpu_sc as plsc

EMB = 128
NC, NS = 2, 16
NW = NC * NS
CHUNK = 128


NBUF = 5


def _sc_gather(table, idx_mat):
    cw = idx_mat.shape[1]
    n_chunks = NW * cw
    p = cw // NBUF
    assert cw % NBUF == 0
    mesh = plsc.VectorSubcoreMesh(core_axis_name="c", subcore_axis_name="s")

    @functools.partial(
        pl.kernel,
        out_type=jax.ShapeDtypeStruct((n_chunks * CHUNK, EMB), jnp.float32),
        mesh=mesh,
        scratch_types=(
            [pltpu.VMEM((cw, CHUNK), jnp.int32)]
            + [pltpu.VMEM((CHUNK, EMB), jnp.float32) for _ in range(NBUF)]
            + [pltpu.SemaphoreType.DMA for _ in range(2 * NBUF)]
        ),
    )
    def gather_k(table_hbm, idx_hbm, out_hbm, idx_v, *rest):
        rows = rest[:NBUF]
        gsem = rest[NBUF:2 * NBUF]
        wsem = rest[2 * NBUF:]
        wid = lax.axis_index("s") * NC + lax.axis_index("c")
        cbase = wid * cw
        rbase = cbase * CHUNK

        pltpu.sync_copy(idx_hbm.at[wid], idx_v)
        for b in range(NBUF):
            pltpu.async_copy(table_hbm.at[idx_v.at[b]], rows[b], gsem[b])

        def body(i, carry):
            for b in range(NBUF):
                c = i * NBUF + b
                pltpu.make_async_copy(table_hbm.at[idx_v.at[c]], rows[b],
                                      gsem[b]).wait()
                pltpu.async_copy(
                    rows[b], out_hbm.at[pl.ds(rbase + c * CHUNK, CHUNK)],
                    wsem[b])

            @pl.when(i < p - 1)
            def _():
                for b in range(NBUF):
                    c2 = (i + 1) * NBUF + b
                    pltpu.make_async_copy(
                        rows[b], out_hbm.at[pl.ds(rbase, CHUNK)],
                        wsem[b]).wait()
                    pltpu.async_copy(table_hbm.at[idx_v.at[c2]], rows[b],
                                     gsem[b])
            return carry

        lax.fori_loop(0, p, body, 0)
        for b in range(NBUF):
            pltpu.make_async_copy(rows[b], out_hbm.at[pl.ds(rbase, CHUNK)],
                                  wsem[b]).wait()

    return gather_k(table, idx_mat)


def _mish(x):
    u = jnp.exp(jnp.minimum(x, 30.0))
    v = u * (u + 2.0)
    return x * (v / (v + 2.0))


def _mlp_block(nt, d, tb, view, off_rows, wi_t, bi, wo_t, bo):
    def body(x_ref, wi_ref, bi_ref, wo_ref, bo_ref, out_ref):
        x = x_ref[...]
        h = _mish(jnp.dot(x, wi_ref[...], preferred_element_type=jnp.float32)
                  + bi_ref[...])
        out_ref[...] = (
            x + jnp.dot(h, wo_ref[...], preferred_element_type=jnp.float32)
            + bo_ref[...])

    grid = nt // tb
    in_specs = [
        pl.BlockSpec((tb, d), lambda i, o=off_rows // tb: (o + i, 0)),
        pl.BlockSpec((d, d), lambda i: (0, 0)),
        pl.BlockSpec((1, d), lambda i: (0, 0)),
        pl.BlockSpec((d, d), lambda i: (0, 0)),
        pl.BlockSpec((1, d), lambda i: (0, 0)),
    ]
    return pl.pallas_call(
        body,
        grid=(grid,),
        in_specs=in_specs,
        out_specs=pl.BlockSpec((tb, d), lambda i: (i, 0)),
        out_shape=jax.ShapeDtypeStruct((nt, d), jnp.float32),
        compiler_params=pltpu.CompilerParams(
            dimension_semantics=("arbitrary",)),
    )(view, wi_t, bi, wo_t, bo)


def kernel(node_embeddings, rel_unary_idx, rel_binary_idx, rel_ternary_idx,
           W1_inner, b1_inner, W1_outer, b1_outer,
           W2_inner, b2_inner, W2_outer, b2_outer,
           W3_inner, b3_inner, W3_outer, b3_outer):
    n1 = rel_unary_idx.shape[0]
    n2 = rel_binary_idx.shape[0] // 2
    n3 = rel_ternary_idx.shape[0] // 3
    tb = 1000

    s3_raw = n1 + 2 * n2
    s3 = ((s3_raw + 3 * tb - 1) // (3 * tb)) * (3 * tb)
    total = s3 + 3 * n3
    m = NW * CHUNK * NBUF
    total_pad = ((total + m - 1) // m) * m
    idx_flat = jnp.concatenate([
        rel_unary_idx, rel_binary_idx,
        jnp.zeros((s3 - s3_raw,), rel_unary_idx.dtype),
        rel_ternary_idx,
        jnp.zeros((total_pad - total,), rel_unary_idx.dtype)])
    g = _sc_gather(node_embeddings, idx_flat.reshape(NW, -1, CHUNK))

    o1 = _mlp_block(n1, EMB, tb, g, 0,
                    W1_inner.T, b1_inner.reshape(1, -1),
                    W1_outer.T, b1_outer.reshape(1, -1))
    o2 = _mlp_block(n2, 2 * EMB, tb, g.reshape(-1, 2 * EMB), n1 // 2,
                    W2_inner.T, b2_inner.reshape(1, -1),
                    W2_outer.T, b2_outer.reshape(1, -1))
    o3 = _mlp_block(n3, 3 * EMB, tb, g.reshape(-1, 3 * EMB), s3 // 3,
                    W3_inner.T, b3_inner.reshape(1, -1),
                    W3_outer.T, b3_outer.reshape(1, -1))

    output_messages = jnp.concatenate(
        [o1, o2.reshape(-1, EMB), o3.reshape(-1, EMB)], axis=0)
    output_indices = jnp.concatenate(
        [rel_unary_idx, rel_binary_idx, rel_ternary_idx], axis=0)
    return (output_messages, output_indices)


# speedup vs baseline: 1.6038x; 1.6038x over previous
import functools

import jax
import jax.numpy as jnp
from jax import lax
from jax.experimental import pallas as pl
from jax.experimental.pallas import tpu as pltpu
from jax.experimental.pallas import tpu_sc as plsc

EMB = 128
NC, NS = 2, 16
NW = NC * NS
CHUNK = 128


NBUF = 5


def _sc_gather(table, idx_mat, s2, s3, total):
    cw = idx_mat.shape[2] // CHUNK
    p = cw // NBUF
    assert cw % NBUF == 0 and cw % 3 == 0
    cu, cb = s2 // CHUNK, s3 // CHUNK
    mesh = plsc.VectorSubcoreMesh(core_axis_name="c", subcore_axis_name="s")
    L = 16

    @functools.partial(
        pl.kernel,
        out_type=[jax.ShapeDtypeStruct((s2, EMB), jnp.float32),
                  jax.ShapeDtypeStruct(((s3 - s2) // 2, 2 * EMB), jnp.float32),
                  jax.ShapeDtypeStruct(((total - s3) // 3, 3 * EMB),
                                       jnp.float32)],
        mesh=mesh,
        scratch_types=(
            [pltpu.VMEM((cw * CHUNK,), jnp.int32)]
            + [pltpu.VMEM((CHUNK, EMB), jnp.float32) for _ in range(NBUF)]
            + [pltpu.SemaphoreType.DMA for _ in range(2 * NBUF)]
        ),
    )
    def gather_k(table_hbm, idx_hbm, g1, g2, g3, idx_v, *rest):
        rows = rest[:NBUF]
        gsem = rest[NBUF:2 * NBUF]
        wsem = rest[2 * NBUF:]
        wid = lax.axis_index("s") * NC + lax.axis_index("c")
        cbase = wid * cw

        def fire_gather(c, b):
            pltpu.async_copy(table_hbm.at[idx_v.at[pl.ds(c * CHUNK, CHUNK)]],
                             rows[b], gsem[b])

        def fire_writeback(c, b):
            g = cbase + c

            @pl.when(g < cu)
            def _():
                pltpu.async_copy(rows[b], g1.at[pl.ds(g * CHUNK, CHUNK)],
                                 wsem[b])

            @pl.when(jnp.logical_and(g >= cu, g < cb))
            def _():
                t0 = pl.multiple_of((g - cu) * 64, 64)
                pltpu.async_copy(rows[b].at[pl.ds(0, 64)],
                                 g2.at[pl.ds(t0, 64), pl.ds(0, EMB)], wsem[b])
                pltpu.async_copy(rows[b].at[pl.ds(64, 64)],
                                 g2.at[pl.ds(t0, 64), pl.ds(EMB, EMB)],
                                 wsem[b])

            @pl.when(g >= cb)
            def _():
                p3 = lax.rem(g - cb, 3)
                t0 = pl.multiple_of((g - cb - p3) // 3 * CHUNK, CHUNK)
                col = pl.multiple_of(p3 * EMB, EMB)
                pltpu.async_copy(rows[b],
                                 g3.at[pl.ds(t0, CHUNK), pl.ds(col, EMB)],
                                 wsem[b])

        def wait_gather(b):
            pltpu.make_async_copy(table_hbm.at[idx_v.at[pl.ds(0, CHUNK)]],
                                  rows[b], gsem[b]).wait()

        def wait_writeback(b):
            pltpu.make_async_copy(rows[b], g1.at[pl.ds(0, CHUNK)],
                                  wsem[b]).wait()

        pltpu.sync_copy(idx_hbm.at[wid, 0], idx_v)
        for b in range(NBUF):
            fire_gather(b, b)

        def body(i, carry):
            for b in range(NBUF):
                wait_gather(b)
                fire_writeback(i * NBUF + b, b)

            @pl.when(i < p - 1)
            def _():
                for b in range(NBUF):
                    wait_writeback(b)
                    fire_gather((i + 1) * NBUF + b, b)
            return carry

        lax.fori_loop(0, p, body, 0)
        for b in range(NBUF):
            wait_writeback(b)

    return gather_k(table, idx_mat)


def _mish(x):
    u = jnp.exp(jnp.minimum(x, 30.0))
    v = u * (u + 2.0)
    return x * (v / (v + 2.0))


def _mlp_block(nt, arity, tb, gathered, wi_t, bi, wo_t, bo):
    d = arity * EMB

    def body(x_ref, wi_ref, bi_ref, wo_ref, bo_ref, out_ref):
        x = x_ref[...]
        h = _mish(jnp.dot(x, wi_ref[...], preferred_element_type=jnp.float32)
                  + bi_ref[...])
        o = (x + jnp.dot(h, wo_ref[...], preferred_element_type=jnp.float32)
             + bo_ref[...])
        out_ref[...] = o.reshape(tb * arity, EMB)

    grid = nt // tb
    in_specs = [
        pl.BlockSpec((tb, d), lambda i: (i, 0)),
        pl.BlockSpec((d, d), lambda i: (0, 0)),
        pl.BlockSpec((1, d), lambda i: (0, 0)),
        pl.BlockSpec((d, d), lambda i: (0, 0)),
        pl.BlockSpec((1, d), lambda i: (0, 0)),
    ]
    return pl.pallas_call(
        body,
        grid=(grid,),
        in_specs=in_specs,
        out_specs=pl.BlockSpec((tb * arity, EMB), lambda i: (i, 0)),
        out_shape=jax.ShapeDtypeStruct((nt * arity, EMB), jnp.float32),
        compiler_params=pltpu.CompilerParams(
            dimension_semantics=("arbitrary",)),
    )(gathered, wi_t, bi, wo_t, bo)


def kernel(node_embeddings, rel_unary_idx, rel_binary_idx, rel_ternary_idx,
           W1_inner, b1_inner, W1_outer, b1_outer,
           W2_inner, b2_inner, W2_outer, b2_outer,
           W3_inner, b3_inner, W3_outer, b3_outer):
    n1 = rel_unary_idx.shape[0]
    n2 = rel_binary_idx.shape[0] // 2
    n3 = rel_ternary_idx.shape[0] // 3
    tb = 1000

    s2 = ((n1 + CHUNK - 1) // CHUNK) * CHUNK
    s3 = ((s2 + 2 * n2 + 3 * CHUNK - 1) // (3 * CHUNK)) * (3 * CHUNK)
    m = NW * CHUNK * 3 * NBUF
    total = ((s3 + 3 * n3 + m - 1) // m) * m
    dt = rel_unary_idx.dtype
    idx_b = jnp.concatenate([rel_binary_idx,
                             jnp.zeros((s3 - s2 - 2 * n2,), dt)])
    idx_b = idx_b.reshape(-1, 64, 2).transpose(0, 2, 1).reshape(-1)
    idx_t = jnp.concatenate([rel_ternary_idx,
                             jnp.zeros((total - s3 - 3 * n3,), dt)])
    idx_t = idx_t.reshape(-1, CHUNK, 3).transpose(0, 2, 1).reshape(-1)
    idx_flat = jnp.concatenate([
        rel_unary_idx, jnp.zeros((s2 - n1,), dt), idx_b, idx_t])
    g1, g2, g3 = _sc_gather(node_embeddings,
                            idx_flat.reshape(NW, 1, -1), s2, s3, total)

    o1 = _mlp_block(n1, 1, tb, g1,
                    W1_inner.T, b1_inner.reshape(1, -1),
                    W1_outer.T, b1_outer.reshape(1, -1))
    o2 = _mlp_block(n2, 2, tb, g2,
                    W2_inner.T, b2_inner.reshape(1, -1),
                    W2_outer.T, b2_outer.reshape(1, -1))
    o3 = _mlp_block(n3, 3, tb, g3,
                    W3_inner.T, b3_inner.reshape(1, -1),
                    W3_outer.T, b3_outer.reshape(1, -1))

    output_messages = jnp.concatenate([o1, o2, o3], axis=0)
    output_indices = jnp.concatenate(
        [rel_unary_idx, rel_binary_idx, rel_ternary_idx], axis=0)
    return (output_messages, output_indices)


# speedup vs baseline: 1.9075x; 1.1894x over previous
import functools

import jax
import jax.numpy as jnp
from jax import lax
from jax.experimental import pallas as pl
from jax.experimental.pallas import tpu as pltpu
from jax.experimental.pallas import tpu_sc as plsc

EMB = 128
NC, NS = 2, 16
NW = NC * NS
CHUNK = 128


NBUF = 5


def _sc_gather(table, idx_mat, s2, s3, total):
    cw = idx_mat.shape[2] // CHUNK
    p = cw // NBUF
    assert cw % NBUF == 0
    cu, cb = s2 // CHUNK, s3 // CHUNK
    mesh = plsc.VectorSubcoreMesh(core_axis_name="c", subcore_axis_name="s")
    L = 16

    @functools.partial(
        pl.kernel,
        out_type=[jax.ShapeDtypeStruct((s2, EMB), jnp.float32),
                  jax.ShapeDtypeStruct((s3 - s2, EMB), jnp.float32),
                  jax.ShapeDtypeStruct((total - s3, EMB), jnp.float32)],
        mesh=mesh,
        scratch_types=(
            [pltpu.VMEM((cw * CHUNK,), jnp.int32)]
            + [pltpu.VMEM((CHUNK, EMB), jnp.float32) for _ in range(NBUF)]
            + [pltpu.SemaphoreType.DMA for _ in range(2 * NBUF)]
        ),
    )
    def gather_k(table_hbm, idx_hbm, g1, g2, g3, idx_v, *rest):
        rows = rest[:NBUF]
        gsem = rest[NBUF:2 * NBUF]
        wsem = rest[2 * NBUF:]
        wid = lax.axis_index("s") * NC + lax.axis_index("c")
        cbase = wid * cw

        def fire_gather(c, b):
            pltpu.async_copy(table_hbm.at[idx_v.at[pl.ds(c * CHUNK, CHUNK)]],
                             rows[b], gsem[b])

        def fire_writeback(c, b):
            g = cbase + c

            @pl.when(g < cu)
            def _():
                pltpu.async_copy(rows[b], g1.at[pl.ds(g * CHUNK, CHUNK)],
                                 wsem[b])

            @pl.when(jnp.logical_and(g >= cu, g < cb))
            def _():
                pltpu.async_copy(rows[b],
                                 g2.at[pl.ds((g - cu) * CHUNK, CHUNK)],
                                 wsem[b])

            @pl.when(g >= cb)
            def _():
                pltpu.async_copy(rows[b],
                                 g3.at[pl.ds((g - cb) * CHUNK, CHUNK)],
                                 wsem[b])

        def wait_gather(b):
            pltpu.make_async_copy(table_hbm.at[idx_v.at[pl.ds(0, CHUNK)]],
                                  rows[b], gsem[b]).wait()

        def wait_writeback(b):
            pltpu.make_async_copy(rows[b], g1.at[pl.ds(0, CHUNK)],
                                  wsem[b]).wait()

        pltpu.sync_copy(idx_hbm.at[wid, 0], idx_v)
        for b in range(NBUF):
            fire_gather(b, b)

        def body(i, carry):
            for b in range(NBUF):
                wait_gather(b)
                fire_writeback(i * NBUF + b, b)

            @pl.when(i < p - 1)
            def _():
                for b in range(NBUF):
                    wait_writeback(b)
                    fire_gather((i + 1) * NBUF + b, b)
            return carry

        lax.fori_loop(0, p, body, 0)
        for b in range(NBUF):
            wait_writeback(b)

    return gather_k(table, idx_mat)


def _mish(x):
    u = jnp.exp(jnp.minimum(x, 30.0))
    v = u * (u + 2.0)
    return x * (v / (v + 2.0))


def _mlp_block(nt, arity, tb, gathered, wi_t, bi, wo_t, bo):
    d = arity * EMB

    def body(x_ref, wi_ref, bi_ref, wo_ref, bo_ref, out_ref):
        x = x_ref[...].reshape(tb, d)
        h = _mish(jnp.dot(x, wi_ref[...], preferred_element_type=jnp.float32)
                  + bi_ref[...])
        o = (x + jnp.dot(h, wo_ref[...], preferred_element_type=jnp.float32)
             + bo_ref[...])
        out_ref[...] = o.reshape(tb * arity, EMB)

    grid = nt // tb
    in_specs = [
        pl.BlockSpec((tb * arity, EMB), lambda i: (i, 0)),
        pl.BlockSpec((d, d), lambda i: (0, 0)),
        pl.BlockSpec((1, d), lambda i: (0, 0)),
        pl.BlockSpec((d, d), lambda i: (0, 0)),
        pl.BlockSpec((1, d), lambda i: (0, 0)),
    ]
    return pl.pallas_call(
        body,
        grid=(grid,),
        in_specs=in_specs,
        out_specs=pl.BlockSpec((tb * arity, EMB), lambda i: (i, 0)),
        out_shape=jax.ShapeDtypeStruct((nt * arity, EMB), jnp.float32),
        compiler_params=pltpu.CompilerParams(
            dimension_semantics=("arbitrary",)),
    )(gathered, wi_t, bi, wo_t, bo)


def kernel(node_embeddings, rel_unary_idx, rel_binary_idx, rel_ternary_idx,
           W1_inner, b1_inner, W1_outer, b1_outer,
           W2_inner, b2_inner, W2_outer, b2_outer,
           W3_inner, b3_inner, W3_outer, b3_outer):
    n1 = rel_unary_idx.shape[0]
    n2 = rel_binary_idx.shape[0] // 2
    n3 = rel_ternary_idx.shape[0] // 3
    tb = 1000

    s2 = ((n1 + CHUNK - 1) // CHUNK) * CHUNK
    s3 = ((s2 + 2 * n2 + CHUNK - 1) // CHUNK) * CHUNK
    m = NW * CHUNK * NBUF
    total = ((s3 + 3 * n3 + m - 1) // m) * m
    dt = rel_unary_idx.dtype
    idx_flat = jnp.concatenate([
        rel_unary_idx, jnp.zeros((s2 - n1,), dt),
        rel_binary_idx, jnp.zeros((s3 - s2 - 2 * n2,), dt),
        rel_ternary_idx, jnp.zeros((total - s3 - 3 * n3,), dt)])
    g1, g2, g3 = _sc_gather(node_embeddings,
                            idx_flat.reshape(NW, 1, -1), s2, s3, total)

    o1 = _mlp_block(n1, 1, tb, g1,
                    W1_inner.T, b1_inner.reshape(1, -1),
                    W1_outer.T, b1_outer.reshape(1, -1))
    o2 = _mlp_block(n2, 2, tb, g2,
                    W2_inner.T, b2_inner.reshape(1, -1),
                    W2_outer.T, b2_outer.reshape(1, -1))
    o3 = _mlp_block(n3, 3, tb, g3,
                    W3_inner.T, b3_inner.reshape(1, -1),
                    W3_outer.T, b3_outer.reshape(1, -1))

    output_messages = jnp.concatenate([o1, o2, o3], axis=0)
    output_indices = jnp.concatenate(
        [rel_unary_idx, rel_binary_idx, rel_ternary_idx], axis=0)
    return (output_messages, output_indices)
